# Initial kernel scaffold; baseline (speedup 1.0000x reference)
import jax
import jax.numpy as jnp
from jax.experimental import pallas as pl


def kernel(x, seq_lengths, table):
    raise NotImplementedError("write your pallas kernel here")


# speedup vs baseline: 4.0867x; 4.0867x over previous
import functools

import jax
import jax.numpy as jnp
from jax import lax
from jax.experimental import pallas as pl
from jax.experimental.pallas import tpu as pltpu
from jax.experimental.pallas import tpu_sc as plsc

VOCAB = 100000
EMBED = 64
B = 4096
L = 50
N = B * L

_NC = 2
_NS = 16
_NW = _NC * _NS
_BPW = N // _NW
_CH = 128
_NCH = _BPW // _CH

_mesh = plsc.VectorSubcoreMesh(core_axis_name="c", subcore_axis_name="s")


@functools.partial(
    pl.kernel,
    mesh=_mesh,
    out_type=jax.ShapeDtypeStruct((N, EMBED), jnp.float32),
    scratch_types=[
        pltpu.VMEM((_BPW,), jnp.int32),
        pltpu.VMEM((2, _CH, EMBED), jnp.float32),
        pltpu.SemaphoreType.DMA,
    ],
    compiler_params=pltpu.CompilerParams(use_tc_tiling_on_sc=False),
)
def _gather(idx_hbm, table_hbm, out_hbm, idx_v, rows_v, sem):
    wid = lax.axis_index("s") * _NC + lax.axis_index("c")
    base = wid * _BPW
    pltpu.sync_copy(idx_hbm.at[pl.ds(base, _BPW)], idx_v)

    def body(c, _):
        cbase = c * _CH
        cp = pltpu.async_copy(
            table_hbm.at[idx_v.at[pl.ds(cbase, _CH)]], rows_v.at[0], sem
        )
        cp.wait()
        pltpu.sync_copy(rows_v.at[0], out_hbm.at[pl.ds(base + cbase, _CH)])
        return 0

    lax.fori_loop(0, _NCH, body, 0)


def kernel(x, seq_lengths, table):
    del seq_lengths
    flat = _gather(x.reshape(-1).astype(jnp.int32), table)
    return flat.reshape(B, L, EMBED)


# speedup vs baseline: 4.6108x; 1.1282x over previous
import functools

import jax
import jax.numpy as jnp
from jax import lax
from jax.experimental import pallas as pl
from jax.experimental.pallas import tpu as pltpu
from jax.experimental.pallas import tpu_sc as plsc

VOCAB = 100000
EMBED = 64
B = 4096
L = 50
N = B * L

_NC = 2
_NS = 16
_NW = _NC * _NS
_BPW = N // _NW
_CH = 128
_K = 5
_NGRP = _BPW // (_CH * _K)

_mesh = plsc.VectorSubcoreMesh(core_axis_name="c", subcore_axis_name="s")


@functools.partial(
    pl.kernel,
    mesh=_mesh,
    out_type=jax.ShapeDtypeStruct((N, EMBED), jnp.float32),
    scratch_types=[
        pltpu.VMEM((_BPW,), jnp.int32),
        pltpu.VMEM((2, _K, _CH, EMBED), jnp.float32),
        pltpu.SemaphoreType.DMA,
        pltpu.SemaphoreType.DMA,
    ],
    compiler_params=pltpu.CompilerParams(use_tc_tiling_on_sc=False),
)
def _gather(idx_hbm, table_hbm, out_hbm, idx_v, rows_v, gsem, osem):
    wid = lax.axis_index("s") * _NC + lax.axis_index("c")
    base = wid * _BPW
    pltpu.sync_copy(idx_hbm.at[pl.ds(base, _BPW)], idx_v)

    def gfire(g, s):
        for j in range(_K):
            c = g * _K + j
            pltpu.async_copy(
                table_hbm.at[idx_v.at[pl.ds(c * _CH, _CH)]],
                rows_v.at[s, j],
                gsem,
            )

    def ofire(g, s):
        for j in range(_K):
            c = g * _K + j
            pltpu.async_copy(
                rows_v.at[s, j],
                out_hbm.at[pl.ds(base + c * _CH, _CH)],
                osem,
            )

    def gwait():
        for _ in range(_K):
            pltpu.make_async_copy(
                table_hbm.at[pl.ds(0, _CH)], rows_v.at[0, 0], gsem
            ).wait()

    def owait():
        for _ in range(_K):
            pltpu.make_async_copy(
                rows_v.at[0, 0], out_hbm.at[pl.ds(base, _CH)], osem
            ).wait()

    gfire(0, 0)

    def body(g, _):
        s = g % 2
        gwait()

        @pl.when(g > 0)
        def _():
            owait()

        @pl.when(g + 1 < _NGRP)
        def _():
            gfire(g + 1, 1 - s)

        ofire(g, s)
        return 0

    lax.fori_loop(0, _NGRP, body, 0)
    owait()


def kernel(x, seq_lengths, table):
    del seq_lengths
    flat = _gather(x.reshape(-1).astype(jnp.int32), table)
    return flat.reshape(B, L, EMBED)
